# baseline (device time: 78142 ns/iter reference)
import jax
import jax.numpy as jnp
from jax import lax
from jax.experimental import pallas as pl
from jax.experimental.pallas import tpu as pltpu


def kernel(x, dest):
    n, d = x.shape
    me = lax.axis_index("x")

    not_mine = (dest != me).astype(jnp.int32)
    perm = jnp.argsort(not_mine, stable=True)
    xs = x[perm].astype(jnp.bfloat16)
    keep = n - jnp.sum(not_mine)

    def body(x_ref, out_ref, send_sem, recv_sem):
        my_x = lax.axis_index("x")
        my_y = lax.axis_index("y")
        partner = (1 - my_x, my_y)

        barrier = pltpu.get_barrier_semaphore()
        pl.semaphore_signal(
            barrier, inc=1, device_id=partner,
            device_id_type=pl.DeviceIdType.MESH,
        )
        pl.semaphore_wait(barrier, 1)

        rdma = pltpu.make_async_remote_copy(
            src_ref=x_ref,
            dst_ref=out_ref,
            send_sem=send_sem,
            recv_sem=recv_sem,
            device_id=partner,
            device_id_type=pl.DeviceIdType.MESH,
        )
        rdma.start()
        rdma.wait()

    partner_buf = pl.pallas_call(
        body,
        out_shape=jax.ShapeDtypeStruct((n, d), jnp.bfloat16),
        in_specs=[pl.BlockSpec(memory_space=pltpu.VMEM)],
        out_specs=pl.BlockSpec(memory_space=pltpu.VMEM),
        scratch_shapes=[
            pltpu.SemaphoreType.DMA,
            pltpu.SemaphoreType.DMA,
        ],
        compiler_params=pltpu.CompilerParams(collective_id=0),
    )(xs)

    i = jnp.arange(n)
    k = n - keep
    idx0 = jnp.where(i < keep, i, n + i)
    idx1 = jnp.where(i < k, n + keep + i, i + keep - n)
    idx = jnp.where(me == 0, idx0, idx1)
    both = jnp.concatenate([xs, partner_buf], axis=0)
    return jnp.take(both, idx, axis=0)


# device time: 47358 ns/iter; 1.6500x vs baseline; 1.6500x over previous
import jax
import jax.numpy as jnp
from jax import lax
from jax.experimental import pallas as pl
from jax.experimental.pallas import tpu as pltpu

N_ROWS = 2048
C = 128
PAD = N_ROWS + C
MAX_CHUNKS = PAD // C


def kernel(x, dest):
    n, d = x.shape
    me = lax.axis_index("x")

    perm = jnp.argsort(dest, stable=True)
    perm_pad = jnp.concatenate([perm, jnp.zeros((PAD - n,), perm.dtype)])
    xs = x[perm_pad].astype(jnp.bfloat16)
    kp = jnp.sum((dest == me).astype(jnp.int32))

    def body(kp_ref, x_ref, out_ref, b_ref, send_sems, recv_sems):
        my_x = lax.axis_index("x")
        my_y = lax.axis_index("y")
        partner = (1 - my_x, my_y)

        keep = kp_ref[0]
        k = n - keep

        send_base = jnp.where(my_x == 0, keep, 0)
        s_send = lax.rem(send_base, 8)
        src0 = send_base - s_send
        n_send = (s_send + k + C - 1) // C
        s_recv = jnp.where(my_x == 0, 0, lax.rem(keep, 8))
        n_recv = (s_recv + k + C - 1) // C

        barrier = pltpu.get_barrier_semaphore()
        pl.semaphore_signal(
            barrier, inc=1, device_id=partner,
            device_id_type=pl.DeviceIdType.MESH,
        )
        pl.semaphore_wait(barrier, 1)

        def rdma(j):
            return pltpu.make_async_remote_copy(
                src_ref=x_ref.at[pl.ds(pl.multiple_of(src0 + j * C, 8), C), :],
                dst_ref=b_ref.at[pl.ds(j * C, C), :],
                send_sem=send_sems.at[j],
                recv_sem=recv_sems.at[j],
                device_id=partner,
                device_id_type=pl.DeviceIdType.MESH,
            )

        for j in range(MAX_CHUNKS):
            @pl.when(j < n_send)
            def _():
                rdma(j).start()

        for j in range(MAX_CHUNKS):
            @pl.when(j < n_send)
            def _():
                rdma(j).wait_send()

        for j in range(MAX_CHUNKS):
            @pl.when(j < n_recv)
            def _():
                rdma(j).wait_recv()

        mine = x_ref[: n, :]
        b = b_ref[: n, :]
        shift = jnp.where(my_x == 0, keep - s_recv, lax.rem(n - s_recv, n))
        rolled = pltpu.roll(b, shift, axis=0)
        boundary = jnp.where(my_x == 0, keep, k)
        row = lax.broadcasted_iota(jnp.int32, (n, 1), 0)
        first = jnp.where(my_x == 0, mine, rolled)
        second = jnp.where(my_x == 0, rolled, mine)
        out_ref[...] = jnp.where(row < boundary, first, second)

    return pl.pallas_call(
        body,
        out_shape=jax.ShapeDtypeStruct((n, d), jnp.bfloat16),
        in_specs=[
            pl.BlockSpec(memory_space=pltpu.SMEM),
            pl.BlockSpec(memory_space=pltpu.VMEM),
        ],
        out_specs=pl.BlockSpec(memory_space=pltpu.VMEM),
        scratch_shapes=[
            pltpu.VMEM((PAD, d), jnp.bfloat16),
            pltpu.SemaphoreType.DMA((MAX_CHUNKS,)),
            pltpu.SemaphoreType.DMA((MAX_CHUNKS,)),
        ],
        compiler_params=pltpu.CompilerParams(collective_id=0),
    )(kp.reshape(1), xs)
